# baseline (speedup 1.0000x reference)
import functools

import numpy as np
import jax
import jax.numpy as jnp
from jax import lax
from jax.experimental import pallas as pl
from jax.experimental.pallas import tpu as pltpu
from jax.experimental.pallas import tpu_sc as plsc

RES = 0.5
NPTS = 48
ORIGIN = -11.75
E2 = 0.1353352832366127
W = 13
NXY = W * W
NDROW = 2 * NXY
T = 14
NROW_PAD = 384
NCHUNK = NROW_PAD // 128
ZROWS = 3
ACC_ROWS = T * NPTS * NPTS * ZROWS
STRIPE = ACC_ROWS // 16
ZCH = 12
ZCHR = STRIPE // ZCH
B_SC = 2
B_TC = 4
XB = 8


def _pattern() -> np.ndarray:
    p = np.zeros((NROW_PAD,), np.int32)
    i = np.arange(NDROW)
    xy = i // 2
    zb = i % 2
    x = xy // W
    y = xy % W
    p[:NDROW] = (x * NPTS + y) * ZROWS + zb
    return p.reshape(NCHUNK, 128)


_P_HOST = _pattern()


def _splat(vec, j):
    return vec.at[jnp.full((16,), j, jnp.int32)].get(
        mode="promise_in_bounds")


def _sc_body(atoms, pfull, out, recs, dens, srcA, srcB, idxA, idxB, pvm,
             zbuf, acc, semA, semB):
    c = lax.axis_index("c")
    s = lax.axis_index("s")
    lanes = lax.iota(jnp.int32, 16)
    zero16 = jnp.zeros((16,), jnp.float32)

    pltpu.sync_copy(pfull, pvm)

    def zb_body(i, _):
        zbuf[i, :] = zero16
        return ()
    lax.fori_loop(0, ZCHR, zb_body, ())
    for j in range(NROW_PAD - NDROW):
        srcA[NDROW + j, :] = zero16
        srcB[NDROW + j, :] = zero16

    def atom_body(k, carry):
        b = carry
        r0 = recs[k, 0:16]
        cx = _splat(r0, 0)
        cy = _splat(r0, 1)
        cz = _splat(r0, 2)
        r = _splat(r0, 3)
        r2 = r * r
        inv_r2 = 1.0 / r2
        neg2 = -2.0 * inv_r2
        c1 = (4.0 * E2) * inv_r2
        c2 = (12.0 * E2) / r
        q225 = 2.25 * r2

        def start(cv):
            tx = jnp.clip((cv - 3.0 - ORIGIN) * 2.0, -1.0, 40.0)
            ti = tx.astype(jnp.int32)
            ti = ti + jnp.where(ti.astype(jnp.float32) < tx, 1, 0)
            return jnp.clip(ti, 0, NPTS - W)

        ix0 = start(cx)
        iy0 = start(cy)
        iz0 = start(cz)
        zb0 = jnp.where(iz0 >= 16, 1, 0)
        zp = (zb0 * 16).astype(jnp.float32)

        axv = ORIGIN + RES * (ix0 + lanes).astype(jnp.float32)
        dxv = axv - cx
        dx2v = dxv * dxv
        ayv = ORIGIN + RES * (iy0 + lanes).astype(jnp.float32)
        dyv = ayv - cy
        dy2v = dyv * dyv
        lf = lanes.astype(jnp.float32)
        za = ORIGIN + RES * (zp + lf)
        zb_ = ORIGIN + RES * (zp + 16.0 + lf)
        dz2a = (za - cz) * (za - cz)
        dz2b = (zb_ - cz) * (zb_ - cz)
        eza = jnp.exp(neg2 * dz2a)
        ezb = jnp.exp(neg2 * dz2b)

        def density(d2, gauss):
            yi = lax.bitcast_convert_type(d2, jnp.int32)
            yi = 0x5F3759DF - lax.shift_right_arithmetic(yi, 1)
            y = lax.bitcast_convert_type(yi, jnp.float32)
            hw = (d2 * y) * y
            y = y * (1.5 - 0.5 * hw)
            hw = (d2 * y) * y
            y = y * (1.5 - 0.5 * hw)
            d = d2 * y
            q = c1 * d2 - c2 * d + 9.0 * E2
            return jnp.where(d2 < r2, gauss,
                             jnp.where(d2 < q225, q, 0.0))

        def x_body(x, _):
            d2x = _splat(dx2v, x)

            @plsc.parallel_loop(0, W, unroll=2)
            def y_body(y):
                d2xy = d2x + _splat(dy2v, y)
                exy = jnp.exp(neg2 * d2xy)
                i2 = (x * W + y) * 2
                dens[i2, :] = density(d2xy + dz2a, exy * eza)
                dens[i2 + 1, :] = density(d2xy + dz2b, exy * ezb)
            return ()
        lax.fori_loop(0, W, x_body, ())

        off = ix0 * (NPTS * ZROWS) + iy0 * ZROWS + zb0
        wv = recs[k, 4:20]

        bufs = ((srcA, idxA, semA), (srcB, idxB, semB))
        handles = {}
        for t in range(T):
            srcX, idxX, semX = bufs[t % 2]
            if t >= 2:
                for h in handles[t - 2]:
                    h.wait()

            @plsc.parallel_loop(0, NDROW, unroll=8)
            def w_body(i, _srcX=srcX, _w=_splat(wv, t)):
                _srcX[i, :] = _w * dens[i, :]

            offt = off + t * (NPTS * NPTS * ZROWS)

            @plsc.parallel_loop(0, NCHUNK * 8, unroll=2)
            def i_body(v, _idxX=idxX, _offt=offt):
                g = v // 8
                l = v % 8
                _idxX[g, pl.ds(l * 16, 16)] = pvm[g, pl.ds(l * 16, 16)] + _offt

            handles[t] = [
                pltpu.async_copy(srcX.at[pl.ds(g * 128, 128)],
                                 acc.at[idxX.at[g]], semX, add=True)
                for g in range(NCHUNK)]
        for t in (T - 2, T - 1):
            for h in handles[t]:
                h.wait()
        return carry

    def batch_body(bi, _):
        b = c * (B_SC // 2) + bi
        base = s * STRIPE
        hz = [pltpu.async_copy(zbuf, acc.at[pl.ds(base + j * ZCHR, ZCHR)],
                               semA) for j in range(ZCH)]
        hr = pltpu.async_copy(atoms.at[pl.ds(b * 128 + s * 8, 8)], recs,
                              semB)
        for h in hz:
            h.wait()
        hr.wait()
        plsc.subcore_barrier()
        lax.fori_loop(0, 8, atom_body, b)
        plsc.subcore_barrier()
        hf = [pltpu.async_copy(acc.at[pl.ds(base + j * ZCHR, ZCHR)],
                               out.at[b].at[pl.ds(base + j * ZCHR, ZCHR)],
                               semA) for j in range(ZCH)]
        for h in hf:
            h.wait()
        plsc.subcore_barrier()
        return ()
    lax.fori_loop(0, B_SC // 2, batch_body, ())


YZ = NPTS * NPTS


def _tc_body(coords_ref, types_t_ref, radii_ref, out_ref, dyz2_ref, ayz_ref):
    xs = pl.program_id(1)
    cx = coords_ref[0, 0, :]
    r = radii_ref[0, 0, :]
    r2 = r * r
    inv_r2 = 1.0 / r2

    @pl.when(xs == 0)
    def _():
        cy = coords_ref[0, 1, :]
        cz = coords_ref[0, 2, :]
        lin = lax.broadcasted_iota(jnp.int32, (1, YZ), 1)
        ay = ORIGIN + RES * (lin // NPTS).astype(jnp.float32)
        az = ORIGIN + RES * (lin % NPTS).astype(jnp.float32)
        dy = cy[:, None] - ay
        dz = cz[:, None] - az
        d2 = dy * dy + dz * dz
        dyz2_ref[...] = d2
        ayz_ref[...] = jnp.exp(-2.0 * inv_r2[:, None] * d2)

    inv_r = 1.0 / r
    c1 = (4.0 * E2) * inv_r2
    c2 = (12.0 * E2) * inv_r
    c3 = 9.0 * E2
    r2b = r2[:, None]
    xf = xs.astype(jnp.float32)
    dyz2 = dyz2_ref[...]
    ayz = ayz_ref[...]
    types_t = types_t_ref[0]

    for j in range(XB):
        ax = ORIGIN + RES * (xf * XB + j)
        dx = coords_ref[0, 0, :] - ax
        dx2 = dx * dx
        axg = jnp.exp(-2.0 * inv_r2 * dx2)
        d2 = dx2[:, None] + dyz2
        gauss = axg[:, None] * ayz
        d = jnp.sqrt(d2)
        q = (c1[:, None] * d2 - c2[:, None] * d) + c3
        dens = jnp.where(d2 < r2b, gauss, jnp.where(d2 < 2.25 * r2b, q, 0.0))
        mm = jax.lax.dot_general(
            types_t, dens, (((1,), (0,)), ((), ())),
            preferred_element_type=jnp.float32)
        out_ref[0, :, j, :] = mm


def _tc_kernel(coords, types, radii):
    B, N, _ = coords.shape
    coords_t = jnp.swapaxes(coords, 1, 2)
    types_t = jnp.swapaxes(types, 1, 2)
    radii_r = radii[:, None, :]
    out = pl.pallas_call(
        _tc_body,
        grid=(B, NPTS // XB),
        in_specs=[
            pl.BlockSpec((1, 3, N), lambda b, x: (b, 0, 0)),
            pl.BlockSpec((1, T, N), lambda b, x: (b, 0, 0)),
            pl.BlockSpec((1, 1, N), lambda b, x: (b, 0, 0)),
        ],
        out_specs=pl.BlockSpec((1, T, XB, YZ), lambda b, x: (b, 0, x, 0)),
        out_shape=jax.ShapeDtypeStruct((B, T, NPTS, YZ), jnp.float32),
        scratch_shapes=[
            pltpu.VMEM((N, YZ), jnp.float32),
            pltpu.VMEM((N, YZ), jnp.float32),
        ],
    )(coords_t, types_t, radii_r)
    return out.reshape(B, T, NPTS, NPTS, NPTS)


def _sc_kernel(coords, types, radii):
    B, N, _ = coords.shape
    flat = B * N
    atoms = jnp.concatenate(
        [coords.reshape(flat, 3), radii.reshape(flat, 1),
         types.reshape(flat, T),
         jnp.zeros((flat, 32 - 4 - T), jnp.float32)], axis=1)
    pfull = jnp.asarray(_P_HOST)

    mesh = plsc.VectorSubcoreMesh(core_axis_name="c", subcore_axis_name="s")
    run = pl.kernel(
        _sc_body,
        mesh=mesh,
        compiler_params=pltpu.CompilerParams(use_tc_tiling_on_sc=False),
        out_type=jax.ShapeDtypeStruct((B, ACC_ROWS, 16), jnp.float32),
        scratch_types=[
            pltpu.VMEM((8, 32), jnp.float32),
            pltpu.VMEM((NDROW, 16), jnp.float32),
            pltpu.VMEM((NROW_PAD, 16), jnp.float32),
            pltpu.VMEM((NROW_PAD, 16), jnp.float32),
            pltpu.VMEM((NCHUNK, 128), jnp.int32),
            pltpu.VMEM((NCHUNK, 128), jnp.int32),
            pltpu.VMEM((NCHUNK, 128), jnp.int32),
            pltpu.VMEM((ZCHR, 16), jnp.float32),
            pltpu.VMEM_SHARED((ACC_ROWS, 16), jnp.float32),
            pltpu.SemaphoreType.DMA,
            pltpu.SemaphoreType.DMA,
        ],
    )
    out = run(atoms, pfull)
    return out.reshape(B, T, NPTS, NPTS, NPTS)


@jax.jit
def kernel(coords, types, radii):
    sc_out = _sc_kernel(coords[:B_SC], types[:B_SC], radii[:B_SC])
    tc_out = _tc_kernel(coords[B_SC:], types[B_SC:], radii[B_SC:])
    return jnp.concatenate([sc_out, tc_out], axis=0)


# speedup vs baseline: 1.2128x; 1.2128x over previous
import functools

import numpy as np
import jax
import jax.numpy as jnp
from jax import lax
from jax.experimental import pallas as pl
from jax.experimental.pallas import tpu as pltpu
from jax.experimental.pallas import tpu_sc as plsc

RES = 0.5
NPTS = 48
ORIGIN = -11.75
E2 = 0.1353352832366127
W = 13
NXY = W * W
NDROW = 2 * NXY
T = 14
NROW_PAD = 384
NCHUNK = NROW_PAD // 128
ZROWS = 3
ACC_ROWS = T * NPTS * NPTS * ZROWS
STRIPE = ACC_ROWS // 16
ZCH = 12
ZCHR = STRIPE // ZCH
B_SC = 4
B_TC = 4
XB = 8


def _pattern() -> np.ndarray:
    p = np.zeros((NROW_PAD,), np.int32)
    i = np.arange(NDROW)
    xy = i // 2
    zb = i % 2
    x = xy // W
    y = xy % W
    p[:NDROW] = (x * NPTS + y) * ZROWS + zb
    return p.reshape(NCHUNK, 128)


_P_HOST = _pattern()


def _splat(vec, j):
    return vec.at[jnp.full((16,), j, jnp.int32)].get(
        mode="promise_in_bounds")


def _sc_body(atoms, pfull, out, recs, dens, srcA, srcB, idxA, idxB, pvm,
             zbuf, acc, semA, semB):
    c = lax.axis_index("c")
    s = lax.axis_index("s")
    lanes = lax.iota(jnp.int32, 16)
    zero16 = jnp.zeros((16,), jnp.float32)

    pltpu.sync_copy(pfull, pvm)

    def zb_body(i, _):
        zbuf[i, :] = zero16
        return ()
    lax.fori_loop(0, ZCHR, zb_body, ())
    for j in range(NROW_PAD - NDROW):
        srcA[NDROW + j, :] = zero16
        srcB[NDROW + j, :] = zero16

    def atom_body(k, carry):
        b = carry
        r0 = recs[k, 0:16]
        cx = _splat(r0, 0)
        cy = _splat(r0, 1)
        cz = _splat(r0, 2)
        r = _splat(r0, 3)
        r2 = r * r
        inv_r2 = 1.0 / r2
        neg2 = -2.0 * inv_r2
        c1 = (4.0 * E2) * inv_r2
        c2 = (12.0 * E2) / r
        q225 = 2.25 * r2

        def start(cv):
            tx = jnp.clip((cv - 3.0 - ORIGIN) * 2.0, -1.0, 40.0)
            ti = tx.astype(jnp.int32)
            ti = ti + jnp.where(ti.astype(jnp.float32) < tx, 1, 0)
            return jnp.clip(ti, 0, NPTS - W)

        ix0 = start(cx)
        iy0 = start(cy)
        iz0 = start(cz)
        zb0 = jnp.where(iz0 >= 16, 1, 0)
        zp = (zb0 * 16).astype(jnp.float32)

        axv = ORIGIN + RES * (ix0 + lanes).astype(jnp.float32)
        dxv = axv - cx
        dx2v = dxv * dxv
        ayv = ORIGIN + RES * (iy0 + lanes).astype(jnp.float32)
        dyv = ayv - cy
        dy2v = dyv * dyv
        lf = lanes.astype(jnp.float32)
        za = ORIGIN + RES * (zp + lf)
        zb_ = ORIGIN + RES * (zp + 16.0 + lf)
        dz2a = (za - cz) * (za - cz)
        dz2b = (zb_ - cz) * (zb_ - cz)
        eza = jnp.exp(neg2 * dz2a)
        ezb = jnp.exp(neg2 * dz2b)

        def density(d2, gauss):
            yi = lax.bitcast_convert_type(d2, jnp.int32)
            yi = 0x5F3759DF - lax.shift_right_arithmetic(yi, 1)
            y = lax.bitcast_convert_type(yi, jnp.float32)
            hw = (d2 * y) * y
            y = y * (1.5 - 0.5 * hw)
            hw = (d2 * y) * y
            y = y * (1.5 - 0.5 * hw)
            d = d2 * y
            q = c1 * d2 - c2 * d + 9.0 * E2
            return jnp.where(d2 < r2, gauss,
                             jnp.where(d2 < q225, q, 0.0))

        def x_body(x, _):
            d2x = _splat(dx2v, x)

            @plsc.parallel_loop(0, W, unroll=2)
            def y_body(y):
                d2xy = d2x + _splat(dy2v, y)
                exy = jnp.exp(neg2 * d2xy)
                i2 = (x * W + y) * 2
                dens[i2, :] = density(d2xy + dz2a, exy * eza)
                dens[i2 + 1, :] = density(d2xy + dz2b, exy * ezb)
            return ()
        lax.fori_loop(0, W, x_body, ())

        off = ix0 * (NPTS * ZROWS) + iy0 * ZROWS + zb0
        wv = recs[k, 4:20]

        bufs = ((srcA, idxA, semA), (srcB, idxB, semB))
        handles = {}
        for t in range(T):
            srcX, idxX, semX = bufs[t % 2]
            if t >= 2:
                for h in handles[t - 2]:
                    h.wait()

            @plsc.parallel_loop(0, NDROW, unroll=8)
            def w_body(i, _srcX=srcX, _w=_splat(wv, t)):
                _srcX[i, :] = _w * dens[i, :]

            offt = off + t * (NPTS * NPTS * ZROWS)

            @plsc.parallel_loop(0, NCHUNK * 8, unroll=2)
            def i_body(v, _idxX=idxX, _offt=offt):
                g = v // 8
                l = v % 8
                _idxX[g, pl.ds(l * 16, 16)] = pvm[g, pl.ds(l * 16, 16)] + _offt

            handles[t] = [
                pltpu.async_copy(srcX.at[pl.ds(g * 128, 128)],
                                 acc.at[idxX.at[g]], semX, add=True)
                for g in range(NCHUNK)]
        for t in (T - 2, T - 1):
            for h in handles[t]:
                h.wait()
        return carry

    def batch_body(bi, _):
        b = c * (B_SC // 2) + bi
        base = s * STRIPE
        hz = [pltpu.async_copy(zbuf, acc.at[pl.ds(base + j * ZCHR, ZCHR)],
                               semA) for j in range(ZCH)]
        hr = pltpu.async_copy(atoms.at[pl.ds(b * 128 + s * 8, 8)], recs,
                              semB)
        for h in hz:
            h.wait()
        hr.wait()
        plsc.subcore_barrier()
        lax.fori_loop(0, 8, atom_body, b)
        plsc.subcore_barrier()
        hf = [pltpu.async_copy(acc.at[pl.ds(base + j * ZCHR, ZCHR)],
                               out.at[b].at[pl.ds(base + j * ZCHR, ZCHR)],
                               semA) for j in range(ZCH)]
        for h in hf:
            h.wait()
        plsc.subcore_barrier()
        return ()
    lax.fori_loop(0, B_SC // 2, batch_body, ())


YZ = NPTS * NPTS


def _tc_body(coords_ref, types_t_ref, radii_ref, out_ref, dyz2_ref, ayz_ref):
    xs = pl.program_id(1)
    cx = coords_ref[0, 0, :]
    r = radii_ref[0, 0, :]
    r2 = r * r
    inv_r2 = 1.0 / r2

    @pl.when(xs == 0)
    def _():
        cy = coords_ref[0, 1, :]
        cz = coords_ref[0, 2, :]
        lin = lax.broadcasted_iota(jnp.int32, (1, YZ), 1)
        ay = ORIGIN + RES * (lin // NPTS).astype(jnp.float32)
        az = ORIGIN + RES * (lin % NPTS).astype(jnp.float32)
        dy = cy[:, None] - ay
        dz = cz[:, None] - az
        d2 = dy * dy + dz * dz
        dyz2_ref[...] = d2
        ayz_ref[...] = jnp.exp(-2.0 * inv_r2[:, None] * d2)

    inv_r = 1.0 / r
    c1 = (4.0 * E2) * inv_r2
    c2 = (12.0 * E2) * inv_r
    c3 = 9.0 * E2
    r2b = r2[:, None]
    xf = xs.astype(jnp.float32)
    dyz2 = dyz2_ref[...]
    ayz = ayz_ref[...]
    types_t = types_t_ref[0]

    for j in range(XB):
        ax = ORIGIN + RES * (xf * XB + j)
        dx = coords_ref[0, 0, :] - ax
        dx2 = dx * dx
        axg = jnp.exp(-2.0 * inv_r2 * dx2)
        d2 = dx2[:, None] + dyz2
        gauss = axg[:, None] * ayz
        d = jnp.sqrt(d2)
        q = (c1[:, None] * d2 - c2[:, None] * d) + c3
        dens = jnp.where(d2 < r2b, gauss, jnp.where(d2 < 2.25 * r2b, q, 0.0))
        mm = jax.lax.dot_general(
            types_t, dens, (((1,), (0,)), ((), ())),
            preferred_element_type=jnp.float32)
        out_ref[0, :, j, :] = mm


def _tc_kernel(coords, types, radii):
    B, N, _ = coords.shape
    coords_t = jnp.swapaxes(coords, 1, 2)
    types_t = jnp.swapaxes(types, 1, 2)
    radii_r = radii[:, None, :]
    out = pl.pallas_call(
        _tc_body,
        grid=(B, NPTS // XB),
        in_specs=[
            pl.BlockSpec((1, 3, N), lambda b, x: (b, 0, 0)),
            pl.BlockSpec((1, T, N), lambda b, x: (b, 0, 0)),
            pl.BlockSpec((1, 1, N), lambda b, x: (b, 0, 0)),
        ],
        out_specs=pl.BlockSpec((1, T, XB, YZ), lambda b, x: (b, 0, x, 0)),
        out_shape=jax.ShapeDtypeStruct((B, T, NPTS, YZ), jnp.float32),
        scratch_shapes=[
            pltpu.VMEM((N, YZ), jnp.float32),
            pltpu.VMEM((N, YZ), jnp.float32),
        ],
    )(coords_t, types_t, radii_r)
    return out.reshape(B, T, NPTS, NPTS, NPTS)


def _sc_kernel(coords, types, radii):
    B, N, _ = coords.shape
    flat = B * N
    atoms = jnp.concatenate(
        [coords.reshape(flat, 3), radii.reshape(flat, 1),
         types.reshape(flat, T),
         jnp.zeros((flat, 32 - 4 - T), jnp.float32)], axis=1)
    pfull = jnp.asarray(_P_HOST)

    mesh = plsc.VectorSubcoreMesh(core_axis_name="c", subcore_axis_name="s")
    run = pl.kernel(
        _sc_body,
        mesh=mesh,
        compiler_params=pltpu.CompilerParams(use_tc_tiling_on_sc=False),
        out_type=jax.ShapeDtypeStruct((B, ACC_ROWS, 16), jnp.float32),
        scratch_types=[
            pltpu.VMEM((8, 32), jnp.float32),
            pltpu.VMEM((NDROW, 16), jnp.float32),
            pltpu.VMEM((NROW_PAD, 16), jnp.float32),
            pltpu.VMEM((NROW_PAD, 16), jnp.float32),
            pltpu.VMEM((NCHUNK, 128), jnp.int32),
            pltpu.VMEM((NCHUNK, 128), jnp.int32),
            pltpu.VMEM((NCHUNK, 128), jnp.int32),
            pltpu.VMEM((ZCHR, 16), jnp.float32),
            pltpu.VMEM_SHARED((ACC_ROWS, 16), jnp.float32),
            pltpu.SemaphoreType.DMA,
            pltpu.SemaphoreType.DMA,
        ],
    )
    out = run(atoms, pfull)
    return out.reshape(B, T, NPTS, NPTS, NPTS)


@jax.jit
def kernel(coords, types, radii):
    sc_out = _sc_kernel(coords[:B_SC], types[:B_SC], radii[:B_SC])
    tc_out = _tc_kernel(coords[B_SC:], types[B_SC:], radii[B_SC:])
    return jnp.concatenate([sc_out, tc_out], axis=0)
